# baseline (speedup 1.0000x reference)
import functools

import jax
import jax.numpy as jnp
from jax import lax
from jax.experimental import pallas as pl
from jax.experimental.pallas import tpu as pltpu
import jax.experimental.pallas.tpu_sc as plsc

N = 10000
E = 320000
D = 128

NC = 2
NS = 16
NW = NC * NS
L = 16
K = 128
C = 80
EPW = C * K
E_PAD = NW * EPW
N_PAD = 10240
RPT = N_PAD // NS
RCH = RPT // K
DEG_W = 16
BLK = N_PAD // 8

_mesh = plsc.VectorSubcoreMesh(
    core_axis_name="c", subcore_axis_name="s", num_cores=NC, num_subcores=NS)


def _deg_body(col_hbm, dp_hbm, col_v, hist_v, sbuf_v, stage_v, shared):
    c = lax.axis_index("c")
    s = lax.axis_index("s")
    w = s * NC + c

    pltpu.sync_copy(col_hbm.at[w], col_v)

    def zero_hist(i, _):
        hist_v[pl.ds(i * L, L)] = jnp.zeros((L,), jnp.float32)
        return 0

    lax.fori_loop(0, N_PAD // L, zero_hist, 0)

    def count(e, _):
        idx = col_v[pl.ds(e * L, L)]
        cnt, last = plsc.scan_count(idx)
        plsc.addupdate_scatter(hist_v, [idx], cnt.astype(jnp.float32),
                               mask=last)
        return 0

    lax.fori_loop(0, EPW // L, count, 0)

    pltpu.sync_copy(hist_v, shared.at[s])
    plsc.subcore_barrier()

    for r in range(NS):
        pltpu.sync_copy(shared.at[r, pl.ds(s * RPT, RPT)], sbuf_v.at[r])

    lanes = lax.iota(jnp.int32, L)

    def combine(v, _):
        acc = jnp.zeros((L,), jnp.float32)
        for r in range(NS):
            acc = acc + sbuf_v[r, pl.ds(v * L, L)]
        plsc.store_scatter(stage_v, [(lanes + v * L) * DEG_W], acc)
        return 0

    lax.fori_loop(0, RPT // L, combine, 0)

    pltpu.sync_copy(stage_v, dp_hbm.at[c, pl.ds(s * RPT * DEG_W, RPT * DEG_W)])


_deg_pass = functools.partial(
    pl.kernel,
    out_type=jax.ShapeDtypeStruct((NC, N_PAD * DEG_W), jnp.float32),
    mesh=_mesh,
    scratch_types=[
        pltpu.VMEM((EPW,), jnp.int32),
        pltpu.VMEM((N_PAD,), jnp.float32),
        pltpu.VMEM((NS, RPT), jnp.float32),
        pltpu.VMEM((RPT * DEG_W,), jnp.float32),
        pltpu.VMEM_SHARED((NS, N_PAD), jnp.float32),
    ],
    compiler_params=pltpu.CompilerParams(needs_layout_passes=False),
)(_deg_body)


NBUF = 2
PH = 32
NCHUNK = E_PAD // K
F0 = 96
F1 = 2 * C - F0


def _agg_body(hs_hbm, col_hbm, row_hbm, agg_hbm, col_q, row_q,
              gb0, gb1, gs0, gs1, ss0, ss1, aggtab):
    c = lax.axis_index("c")
    s = lax.axis_index("s")
    gb = (gb0, gb1)
    gs = (gs0, gs1)
    ss = (ss0, ss1)

    start = jnp.minimum(s * RPT, N - RPT)
    for k in range(RCH):
        b = k % 2
        if k >= 2:
            pltpu.make_async_copy(gb[b], aggtab.at[pl.ds(start, K)],
                                  ss[b]).wait()
        pltpu.async_copy(hs_hbm.at[pl.ds(start + k * K, K)], gb[b],
                         gs[b]).wait()
        pltpu.async_copy(gb[b], aggtab.at[pl.ds(start + k * K, K)], ss[b])
    for b in range(2):
        pltpu.make_async_copy(gb[b], aggtab.at[pl.ds(start, K)], ss[b]).wait()
    plsc.subcore_barrier()

    base = jnp.where(c == 0, s * F0, NS * F0 + s * F1)
    nph = jnp.where(c == 0, F0 // PH, F1 // PH)

    for p in range(F0 // PH):
        @pl.when(p < nph)
        def _():
            pltpu.sync_copy(col_hbm.at[pl.ds(base + p * PH, PH)], col_q)
            pltpu.sync_copy(row_hbm.at[pl.ds(base + p * PH, PH)], row_q)
            for b in range(NBUF):
                pltpu.async_copy(hs_hbm.at[col_q.at[b]], gb[b], gs[b])

            def round_(g, _):
                j0 = g * NBUF
                for b in range(NBUF):
                    pltpu.make_async_copy(hs_hbm.at[col_q.at[0]], gb[b],
                                          gs[b]).wait()
                    pltpu.async_copy(gb[b], aggtab.at[row_q.at[j0 + b]],
                                     ss[b], add=True)
                for b in range(NBUF):
                    pltpu.make_async_copy(gb[b], aggtab.at[row_q.at[0]],
                                          ss[b]).wait()

                    @pl.when(g < PH // NBUF - 1)
                    def _():
                        pltpu.async_copy(hs_hbm.at[col_q.at[j0 + NBUF + b]],
                                         gb[b], gs[b])
                return 0

            lax.fori_loop(0, PH // NBUF, round_, 0)
    plsc.subcore_barrier()

    for k in range(RCH):
        b = k % 2
        if k >= 2:
            pltpu.make_async_copy(gb[b], agg_hbm.at[c, pl.ds(0, K)],
                                  ss[b]).wait()
        pltpu.async_copy(aggtab.at[pl.ds(s * RPT + k * K, K)], gb[b],
                         gs[b]).wait()
        pltpu.async_copy(gb[b], agg_hbm.at[c, pl.ds(s * RPT + k * K, K)],
                         ss[b])
    for b in range(2):
        pltpu.make_async_copy(gb[b], agg_hbm.at[c, pl.ds(0, K)], ss[b]).wait()


def _linear_body(x_ref, w_ref, b_ref, dp_ref, hs_ref):
    deg = dp_ref[0, :, 0:1] + dp_ref[1, :, 0:1] + 1.0
    dinv = lax.rsqrt(deg)
    h = lax.dot_general(
        x_ref[...], w_ref[...],
        (((1,), (1,)), ((), ())),
        preferred_element_type=jnp.float32,
    )
    hs_ref[...] = (h + b_ref[...]) * dinv


def _final_body(agg_ref, hs_ref, dp_ref, out_ref):
    deg = dp_ref[0, :, 0:1] + dp_ref[1, :, 0:1] + 1.0
    dinv = lax.rsqrt(deg)
    out_ref[...] = (agg_ref[0] + agg_ref[1] - hs_ref[...]) * dinv


def kernel(X, edge_index, W, b):
    row = edge_index[0]
    col = edge_index[1]
    pad = jnp.full((E_PAD - E,), N, jnp.int32)
    row4 = jnp.concatenate([row, pad]).reshape(NCHUNK, K)
    colp = jnp.concatenate([col, pad])
    col4 = colp.reshape(NCHUNK, K)
    col2 = colp.reshape(NW, EPW)
    b2 = b.reshape(1, D)

    dp = _deg_pass(col2).reshape(NC, N_PAD, DEG_W)

    hs = pl.pallas_call(
        _linear_body,
        grid=(N_PAD // BLK,),
        in_specs=[
            pl.BlockSpec((BLK, D), lambda i: (i, 0)),
            pl.BlockSpec((D, D), lambda i: (0, 0)),
            pl.BlockSpec((1, D), lambda i: (0, 0)),
            pl.BlockSpec((NC, BLK, DEG_W), lambda i: (0, i, 0)),
        ],
        out_specs=pl.BlockSpec((BLK, D), lambda i: (i, 0)),
        out_shape=jax.ShapeDtypeStruct((N_PAD, D), jnp.float32),
    )(X, W, b2, dp)

    agg = functools.partial(
        pl.kernel,
        out_type=jax.ShapeDtypeStruct((NC, N_PAD, D), jnp.float32),
        mesh=_mesh,
        scratch_types=[
            pltpu.VMEM((PH, K), jnp.int32),
            pltpu.VMEM((PH, K), jnp.int32),
            pltpu.VMEM((K, D), jnp.float32),
            pltpu.VMEM((K, D), jnp.float32),
            pltpu.SemaphoreType.DMA,
            pltpu.SemaphoreType.DMA,
            pltpu.SemaphoreType.DMA,
            pltpu.SemaphoreType.DMA,
            pltpu.VMEM_SHARED((N_PAD, D), jnp.float32),
        ],
    )(_agg_body)(hs, col4, row4)

    out = pl.pallas_call(
        _final_body,
        grid=(N_PAD // BLK,),
        in_specs=[
            pl.BlockSpec((NC, BLK, D), lambda i: (0, i, 0)),
            pl.BlockSpec((BLK, D), lambda i: (i, 0)),
            pl.BlockSpec((NC, BLK, DEG_W), lambda i: (0, i, 0)),
        ],
        out_specs=pl.BlockSpec((BLK, D), lambda i: (i, 0)),
        out_shape=jax.ShapeDtypeStruct((N, D), jnp.float32),
    )(agg, hs, dp)

    return out


# speedup vs baseline: 1.0452x; 1.0452x over previous
import functools

import jax
import jax.numpy as jnp
from jax import lax
from jax.experimental import pallas as pl
from jax.experimental.pallas import tpu as pltpu
import jax.experimental.pallas.tpu_sc as plsc

N = 10000
E = 320000
D = 128

NC = 2
NS = 16
NW = NC * NS
L = 16
K = 128
C = 80
EPW = C * K
E_PAD = NW * EPW
N_PAD = 10240
RPT = N_PAD // NS
RCH = RPT // K
DEG_W = 16
BLK = N_PAD // 8

_mesh = plsc.VectorSubcoreMesh(
    core_axis_name="c", subcore_axis_name="s", num_cores=NC, num_subcores=NS)


def _deg_body(col_hbm, dp_hbm, col_v, hist_v, sbuf_v, stage_v, shared):
    c = lax.axis_index("c")
    s = lax.axis_index("s")
    w = s * NC + c

    pltpu.sync_copy(col_hbm.at[w], col_v)

    def zero_hist(i, _):
        hist_v[pl.ds(i * L, L)] = jnp.zeros((L,), jnp.float32)
        return 0

    lax.fori_loop(0, N_PAD // L, zero_hist, 0)

    def count(e, _):
        idx = col_v[pl.ds(e * L, L)]
        cnt, last = plsc.scan_count(idx)
        plsc.addupdate_scatter(hist_v, [idx], cnt.astype(jnp.float32),
                               mask=last)
        return 0

    lax.fori_loop(0, EPW // L, count, 0)

    pltpu.sync_copy(hist_v, shared.at[s])
    plsc.subcore_barrier()

    for r in range(NS):
        pltpu.sync_copy(shared.at[r, pl.ds(s * RPT, RPT)], sbuf_v.at[r])

    lanes = lax.iota(jnp.int32, L)

    def combine(v, _):
        acc = jnp.zeros((L,), jnp.float32)
        for r in range(NS):
            acc = acc + sbuf_v[r, pl.ds(v * L, L)]
        plsc.store_scatter(stage_v, [(lanes + v * L) * DEG_W], acc)
        return 0

    lax.fori_loop(0, RPT // L, combine, 0)

    pltpu.sync_copy(stage_v, dp_hbm.at[c, pl.ds(s * RPT * DEG_W, RPT * DEG_W)])


_deg_pass = functools.partial(
    pl.kernel,
    out_type=jax.ShapeDtypeStruct((NC, N_PAD * DEG_W), jnp.float32),
    mesh=_mesh,
    scratch_types=[
        pltpu.VMEM((EPW,), jnp.int32),
        pltpu.VMEM((N_PAD,), jnp.float32),
        pltpu.VMEM((NS, RPT), jnp.float32),
        pltpu.VMEM((RPT * DEG_W,), jnp.float32),
        pltpu.VMEM_SHARED((NS, N_PAD), jnp.float32),
    ],
    compiler_params=pltpu.CompilerParams(needs_layout_passes=False),
)(_deg_body)


NBUF = 2
PH = 40
NCHUNK = E_PAD // K
F0 = 120
F1 = 2 * C - F0


def _agg_body(hs_hbm, col_hbm, row_hbm, agg_hbm, col_q, row_q,
              gb0, gb1, gs0, gs1, ss0, ss1, aggtab):
    c = lax.axis_index("c")
    s = lax.axis_index("s")
    gb = (gb0, gb1)
    gs = (gs0, gs1)
    ss = (ss0, ss1)

    start = jnp.minimum(s * RPT, N - RPT)
    for k in range(RCH):
        b = k % 2
        if k >= 2:
            pltpu.make_async_copy(gb[b], aggtab.at[pl.ds(start, K)],
                                  ss[b]).wait()
        pltpu.async_copy(hs_hbm.at[pl.ds(start + k * K, K)], gb[b],
                         gs[b]).wait()
        pltpu.async_copy(gb[b], aggtab.at[pl.ds(start + k * K, K)], ss[b])
    for b in range(2):
        pltpu.make_async_copy(gb[b], aggtab.at[pl.ds(start, K)], ss[b]).wait()
    plsc.subcore_barrier()

    base = jnp.where(c == 0, s * F0, NS * F0 + s * F1)
    nph = jnp.where(c == 0, F0 // PH, F1 // PH)

    for p in range(F0 // PH):
        @pl.when(p < nph)
        def _():
            pltpu.sync_copy(col_hbm.at[pl.ds(base + p * PH, PH)], col_q)
            pltpu.sync_copy(row_hbm.at[pl.ds(base + p * PH, PH)], row_q)
            for b in range(NBUF):
                pltpu.async_copy(hs_hbm.at[col_q.at[b]], gb[b], gs[b])

            def round_(g, _):
                j0 = g * NBUF
                for b in range(NBUF):
                    pltpu.make_async_copy(hs_hbm.at[col_q.at[0]], gb[b],
                                          gs[b]).wait()
                    pltpu.async_copy(gb[b], aggtab.at[row_q.at[j0 + b]],
                                     ss[b], add=True)
                for b in range(NBUF):
                    pltpu.make_async_copy(gb[b], aggtab.at[row_q.at[0]],
                                          ss[b]).wait()

                    @pl.when(g < PH // NBUF - 1)
                    def _():
                        pltpu.async_copy(hs_hbm.at[col_q.at[j0 + NBUF + b]],
                                         gb[b], gs[b])
                return 0

            lax.fori_loop(0, PH // NBUF, round_, 0)
    plsc.subcore_barrier()

    for k in range(RCH):
        b = k % 2
        if k >= 2:
            pltpu.make_async_copy(gb[b], agg_hbm.at[c, pl.ds(0, K)],
                                  ss[b]).wait()
        pltpu.async_copy(aggtab.at[pl.ds(s * RPT + k * K, K)], gb[b],
                         gs[b]).wait()
        pltpu.async_copy(gb[b], agg_hbm.at[c, pl.ds(s * RPT + k * K, K)],
                         ss[b])
    for b in range(2):
        pltpu.make_async_copy(gb[b], agg_hbm.at[c, pl.ds(0, K)], ss[b]).wait()


def _linear_body(x_ref, w_ref, b_ref, dp_ref, hs_ref):
    deg = dp_ref[0, :, 0:1] + dp_ref[1, :, 0:1] + 1.0
    dinv = lax.rsqrt(deg)
    h = lax.dot_general(
        x_ref[...], w_ref[...],
        (((1,), (1,)), ((), ())),
        preferred_element_type=jnp.float32,
    )
    hs_ref[...] = (h + b_ref[...]) * dinv


def _final_body(agg_ref, hs_ref, dp_ref, out_ref):
    deg = dp_ref[0, :, 0:1] + dp_ref[1, :, 0:1] + 1.0
    dinv = lax.rsqrt(deg)
    out_ref[...] = (agg_ref[0] + agg_ref[1] - hs_ref[...]) * dinv


def kernel(X, edge_index, W, b):
    row = edge_index[0]
    col = edge_index[1]
    pad = jnp.full((E_PAD - E,), N, jnp.int32)
    row4 = jnp.concatenate([row, pad]).reshape(NCHUNK, K)
    colp = jnp.concatenate([col, pad])
    col4 = colp.reshape(NCHUNK, K)
    col2 = colp.reshape(NW, EPW)
    b2 = b.reshape(1, D)

    dp = _deg_pass(col2).reshape(NC, N_PAD, DEG_W)

    hs = pl.pallas_call(
        _linear_body,
        grid=(N_PAD // BLK,),
        in_specs=[
            pl.BlockSpec((BLK, D), lambda i: (i, 0)),
            pl.BlockSpec((D, D), lambda i: (0, 0)),
            pl.BlockSpec((1, D), lambda i: (0, 0)),
            pl.BlockSpec((NC, BLK, DEG_W), lambda i: (0, i, 0)),
        ],
        out_specs=pl.BlockSpec((BLK, D), lambda i: (i, 0)),
        out_shape=jax.ShapeDtypeStruct((N_PAD, D), jnp.float32),
    )(X, W, b2, dp)

    agg = functools.partial(
        pl.kernel,
        out_type=jax.ShapeDtypeStruct((NC, N_PAD, D), jnp.float32),
        mesh=_mesh,
        scratch_types=[
            pltpu.VMEM((PH, K), jnp.int32),
            pltpu.VMEM((PH, K), jnp.int32),
            pltpu.VMEM((K, D), jnp.float32),
            pltpu.VMEM((K, D), jnp.float32),
            pltpu.SemaphoreType.DMA,
            pltpu.SemaphoreType.DMA,
            pltpu.SemaphoreType.DMA,
            pltpu.SemaphoreType.DMA,
            pltpu.VMEM_SHARED((N_PAD, D), jnp.float32),
        ],
    )(_agg_body)(hs, col4, row4)

    out = pl.pallas_call(
        _final_body,
        grid=(N_PAD // BLK,),
        in_specs=[
            pl.BlockSpec((NC, BLK, D), lambda i: (0, i, 0)),
            pl.BlockSpec((BLK, D), lambda i: (i, 0)),
            pl.BlockSpec((NC, BLK, DEG_W), lambda i: (0, i, 0)),
        ],
        out_specs=pl.BlockSpec((BLK, D), lambda i: (i, 0)),
        out_shape=jax.ShapeDtypeStruct((N, D), jnp.float32),
    )(agg, hs, dp)

    return out
